# Initial kernel scaffold; baseline (speedup 1.0000x reference)
import jax
import jax.numpy as jnp
from jax.experimental import pallas as pl


def kernel(token_emb, heads_ids, sent_ids, emb_table, W1, b1, W2, b2, W3, b3, conv1_w, conv1_b, conv2_w, conv2_b):
    raise NotImplementedError("write your pallas kernel here")


# speedup vs baseline: 32.2624x; 32.2624x over previous
import jax
import jax.numpy as jnp
from jax import lax
from jax.experimental import pallas as pl
from jax.experimental.pallas import tpu as pltpu

_NT = 2048
_NH = 512
_D = 768
_TT = 32
_TB = _TT + 16
_G = _TB + 8
_DROWS = 127 + 2 * _G + 1
_BTOP = 8
_BROWS = _BTOP + _NT + _TB
_OCOLS = _NT + _TT


def _proj_body(hids_ref, te_ref, w1at_ref, w1bt_ref, w1ct_ref, b1_ref,
               emb_ref, bp_ref, a_ref, dfb_ref, tp_ref):
    f32 = jnp.float32
    bp_ref[pl.ds(0, _BTOP), :] = jnp.zeros((_BTOP, _D), f32)
    bp_ref[pl.ds(_BTOP, _NT), :] = jnp.dot(
        te_ref[...], w1bt_ref[...], preferred_element_type=f32)
    ntail = _BROWS - _BTOP - _NT
    bp_ref[pl.ds(_BTOP + _NT, ntail), :] = jnp.zeros((ntail, _D), f32)
    tp_ref[...] = jnp.dot(te_ref[...], w1at_ref[...], preferred_element_type=f32)
    tok = lax.broadcasted_iota(jnp.int32, (_NH, _NT), 1)
    oh = (hids_ref[...] == tok).astype(f32)
    a_ref[...] = jnp.dot(oh, tp_ref[...], preferred_element_type=f32)
    dfb_ref[...] = jnp.dot(emb_ref[...], w1ct_ref[...],
                           preferred_element_type=f32) + b1_ref[...]


def _span_body(a8s_ref, starts_ref, lens_ref, hids_ref, lmax_ref,
               a_ref, bp_ref, dpad_ref, d127_ref, b1_ref,
               w2t_ref, b2_ref, w3t_ref, b3_ref,
               c1_ref, c1b_ref, c2_ref, c2b_ref, out_ref):
    f32 = jnp.float32
    h = pl.program_id(0)
    a8 = a8s_ref[h]
    sH = starts_ref[h]
    Lh = lens_ref[h]
    hid = hids_ref[h]
    lmax = lmax_ref[0]
    ninf = jnp.float32(-jnp.inf)
    out_ref[...] = jnp.full((1, _OCOLS, 2), ninf, f32)
    nt = lax.div(sH - a8 * 8 + Lh + _TT - 1, _TT)

    def _tile(i, c):
        ob8 = a8 + 4 * i
        tb0 = ob8 * 8 - 8
        bs = bp_ref[pl.ds(ob8 * 8, _TB), :]
        q0 = 63 + tb0 - hid + _G
        q0c = jnp.clip(q0, 0, _DROWS - _G)
        qa = lax.div(q0c, 8)
        s = q0c - qa * 8
        dsl = pltpu.roll(dpad_ref[pl.ds(qa * 8, _TB + 8), :], -s, 0)[0:_TB]
        r = lax.broadcasted_iota(jnp.int32, (_TB, 1), 0)
        jv = tb0 + r - sH
        uv = hid - (tb0 + r) + 63
        dok = (uv >= 0) & (uv <= 126)
        dist = jnp.where(dok, dsl, d127_ref[...])
        pre = a_ref[0] + bs + dist
        pre = jnp.where(jv >= Lh, b1_ref[...], pre)
        x = jnp.maximum(pre, 0.0)
        h2 = jnp.maximum(
            jnp.dot(x, w2t_ref[...], preferred_element_type=f32) + b2_ref[...],
            0.0)
        m = jnp.dot(h2, w3t_ref[...], preferred_element_type=f32) + b3_ref[...]
        m = jnp.where((jv < 0) | (jv >= lmax), 0.0, m)
        y1 = (jnp.dot(m[0:_TB - 2], c1_ref[0:64, :], preferred_element_type=f32)
              + jnp.dot(m[1:_TB - 1], c1_ref[64:128, :], preferred_element_type=f32)
              + jnp.dot(m[2:_TB], c1_ref[128:192, :], preferred_element_type=f32)
              + c1b_ref[...])
        k1 = lax.broadcasted_iota(jnp.int32, (_TB - 2, 1), 0)
        j1 = tb0 + 1 + k1 - sH
        y1 = jnp.where((j1 < 0) | (j1 >= lmax), 0.0, y1)
        y2 = (jnp.dot(y1[6:6 + _TT], c2_ref[0:4, :], preferred_element_type=f32)
              + jnp.dot(y1[7:7 + _TT], c2_ref[4:8, :], preferred_element_type=f32)
              + jnp.dot(y1[8:8 + _TT], c2_ref[8:12, :], preferred_element_type=f32)
              + c2b_ref[...])
        r2 = lax.broadcasted_iota(jnp.int32, (_TT, 1), 0)
        j2 = tb0 + 8 + r2 - sH
        vals = jnp.where((j2 >= 0) & (j2 < Lh), y2, ninf)
        out_ref[0, pl.ds(ob8 * 8, _TT), :] = vals
        return c
    lax.fori_loop(0, nt, _tile, 0)


def kernel(token_emb, heads_ids, sent_ids, emb_table, W1, b1, W2, b2, W3, b3,
           conv1_w, conv1_b, conv2_w, conv2_b):
    f32 = jnp.float32
    heads_ids = heads_ids.astype(jnp.int32)
    sent_ids = sent_ids.astype(jnp.int32)
    w1at = W1[:, :_D].T
    w1bt = W1[:, _D:2 * _D].T
    w1ct = W1[:, 2 * _D:].T
    b1r = b1[None, :]

    bp, a, dfb = pl.pallas_call(
        _proj_body,
        in_specs=[pl.BlockSpec(memory_space=pltpu.VMEM)] * 7,
        out_specs=[pl.BlockSpec(memory_space=pltpu.VMEM)] * 3,
        out_shape=[
            jax.ShapeDtypeStruct((_BROWS, _D), f32),
            jax.ShapeDtypeStruct((_NH, _D), f32),
            jax.ShapeDtypeStruct((128, _D), f32),
        ],
        scratch_shapes=[pltpu.VMEM((_NT, _D), f32)],
    )(heads_ids[:, None], token_emb, w1at, w1bt, w1ct, b1r, emb_table)

    d127 = dfb[127:128]
    dflip = dfb[126::-1]
    dpad = jnp.concatenate(
        [jnp.tile(d127, (_G, 1)), dflip,
         jnp.tile(d127, (_DROWS - _G - 127, 1))], axis=0)

    heads_sent = jnp.take(sent_ids, heads_ids)
    starts = jnp.searchsorted(sent_ids, heads_sent, side='left').astype(jnp.int32)
    ends = jnp.searchsorted(sent_ids, heads_sent, side='right').astype(jnp.int32)
    lens = ends - starts
    a8s = starts // 8
    lmax = jnp.max(lens)[None].astype(jnp.int32)

    w2t = W2.T
    b2r = b2[None, :]
    w3t = W3.T
    b3r = b3[None, :]
    c1 = jnp.concatenate([conv1_w[:, :, k].T for k in range(3)], axis=0)
    c1b = conv1_b[None, :]
    c2 = jnp.concatenate([conv2_w[:, :, k].T for k in range(3)], axis=0)
    c2b = conv2_b[None, :]

    const = lambda h, *_: (0, 0)
    grid_spec = pltpu.PrefetchScalarGridSpec(
        num_scalar_prefetch=5,
        grid=(_NH,),
        in_specs=[
            pl.BlockSpec((1, 1, _D), lambda h, *_: (h, 0, 0)),
            pl.BlockSpec((_BROWS, _D), const),
            pl.BlockSpec((_DROWS, _D), const),
            pl.BlockSpec((1, _D), const),
            pl.BlockSpec((1, _D), const),
            pl.BlockSpec((_D, 256), const),
            pl.BlockSpec((1, 256), const),
            pl.BlockSpec((256, 64), const),
            pl.BlockSpec((1, 64), const),
            pl.BlockSpec((192, 4), const),
            pl.BlockSpec((1, 4), const),
            pl.BlockSpec((12, 2), const),
            pl.BlockSpec((1, 2), const),
        ],
        out_specs=pl.BlockSpec((1, _OCOLS, 2), lambda h, *_: (h, 0, 0)),
    )
    out = pl.pallas_call(
        _span_body,
        grid_spec=grid_spec,
        out_shape=jax.ShapeDtypeStruct((_NH, _OCOLS, 2), f32),
    )(a8s, starts, lens, heads_ids, lmax,
      a[:, None, :], bp, dpad, d127, b1r, w2t, b2r, w3t, b3r,
      c1, c1b, c2, c2b)
    return out[:, :_NT, :]


# speedup vs baseline: 41.6703x; 1.2916x over previous
import jax
import jax.numpy as jnp
from jax import lax
from jax.experimental import pallas as pl
from jax.experimental.pallas import tpu as pltpu

_NT = 2048
_NH = 512
_D = 768
_TT = 32
_TB = _TT + 16
_G = _TB + 8
_DROWS = 127 + 2 * _G + 1
_BTOP = 8
_BROWS = _BTOP + _NT + _TB


def _proj_body(hids_ref, te_ref, w1at_ref, w1bt_ref, w1ct_ref, b1_ref,
               emb_ref, bp_ref, a_ref, dfb_ref, tp_ref):
    f32 = jnp.float32
    bp_ref[pl.ds(0, _BTOP), :] = jnp.zeros((_BTOP, _D), f32)
    bp_ref[pl.ds(_BTOP, _NT), :] = jnp.dot(
        te_ref[...], w1bt_ref[...], preferred_element_type=f32)
    ntail = _BROWS - _BTOP - _NT
    bp_ref[pl.ds(_BTOP + _NT, ntail), :] = jnp.zeros((ntail, _D), f32)
    tp_ref[...] = jnp.dot(te_ref[...], w1at_ref[...], preferred_element_type=f32)
    tok = lax.broadcasted_iota(jnp.int32, (_NH, _NT), 1)
    oh = (hids_ref[...] == tok).astype(f32)
    a_ref[...] = jnp.dot(oh, tp_ref[...], preferred_element_type=f32)
    dfb_ref[...] = jnp.dot(emb_ref[...], w1ct_ref[...],
                           preferred_element_type=f32) + b1_ref[...]


def _span_body(a8s_ref, starts_ref, lens_ref, hids_ref, lmax_ref,
               a_ref, bp_ref, dpad_ref, d127_ref, b1_ref,
               w2t_ref, b2_ref, w3t_ref, b3_ref,
               c1_ref, c1b_ref, c2_ref, c2b_ref, out_ref):
    f32 = jnp.float32
    h = pl.program_id(0)
    a8 = a8s_ref[h]
    sH = starts_ref[h]
    Lh = lens_ref[h]
    hid = hids_ref[h]
    lmax = lmax_ref[0]
    ninf = jnp.float32(-jnp.inf)
    out_ref[...] = jnp.full((1, _NT, 2), ninf, f32)
    nt = lax.div(sH - a8 * 8 + Lh + _TT - 1, _TT)

    def _tile(i, c):
        ob8 = a8 + 4 * i
        tb0 = ob8 * 8 - 8
        bs = bp_ref[pl.ds(ob8 * 8, _TB), :]
        q0 = 63 + tb0 - hid + _G
        q0c = jnp.clip(q0, 0, _DROWS - _G)
        qa = lax.div(q0c, 8)
        s = q0c - qa * 8
        dsl = pltpu.roll(dpad_ref[pl.ds(qa * 8, _TB + 8), :], -s, 0)[0:_TB]
        r = lax.broadcasted_iota(jnp.int32, (_TB, 1), 0)
        jv = tb0 + r - sH
        uv = hid - (tb0 + r) + 63
        dok = (uv >= 0) & (uv <= 126)
        dist = jnp.where(dok, dsl, d127_ref[...])
        pre = a_ref[0] + bs + dist
        pre = jnp.where(jv >= Lh, b1_ref[...], pre)
        x = jnp.maximum(pre, 0.0)
        h2 = jnp.maximum(
            jnp.dot(x, w2t_ref[...], preferred_element_type=f32) + b2_ref[...],
            0.0)
        m = jnp.dot(h2, w3t_ref[...], preferred_element_type=f32) + b3_ref[...]
        m = jnp.where((jv < 0) | (jv >= lmax), 0.0, m)
        y1 = (jnp.dot(m[0:_TB - 2], c1_ref[0:64, :], preferred_element_type=f32)
              + jnp.dot(m[1:_TB - 1], c1_ref[64:128, :], preferred_element_type=f32)
              + jnp.dot(m[2:_TB], c1_ref[128:192, :], preferred_element_type=f32)
              + c1b_ref[...])
        k1 = lax.broadcasted_iota(jnp.int32, (_TB - 2, 1), 0)
        j1 = tb0 + 1 + k1 - sH
        y1 = jnp.where((j1 < 0) | (j1 >= lmax), 0.0, y1)
        y2 = (jnp.dot(y1[6:6 + _TT], c2_ref[0:4, :], preferred_element_type=f32)
              + jnp.dot(y1[7:7 + _TT], c2_ref[4:8, :], preferred_element_type=f32)
              + jnp.dot(y1[8:8 + _TT], c2_ref[8:12, :], preferred_element_type=f32)
              + c2b_ref[...])
        r2 = lax.broadcasted_iota(jnp.int32, (_TT, 1), 0)
        j2 = tb0 + 8 + r2 - sH
        vals = jnp.where((j2 >= 0) & (j2 < Lh), y2, ninf)
        off8 = jnp.minimum(ob8, (_NT - _TT) // 8)
        delta = (ob8 - off8) * 8
        vals = pltpu.roll(vals, delta, 0)
        prev = out_ref[0, pl.ds(off8 * 8, _TT), :]
        out_ref[0, pl.ds(off8 * 8, _TT), :] = jnp.where(r2 >= delta, vals, prev)
        return c
    lax.fori_loop(0, nt, _tile, 0)


def kernel(token_emb, heads_ids, sent_ids, emb_table, W1, b1, W2, b2, W3, b3,
           conv1_w, conv1_b, conv2_w, conv2_b):
    f32 = jnp.float32
    heads_ids = heads_ids.astype(jnp.int32)
    sent_ids = sent_ids.astype(jnp.int32)
    w1at = W1[:, :_D].T
    w1bt = W1[:, _D:2 * _D].T
    w1ct = W1[:, 2 * _D:].T
    b1r = b1[None, :]

    bp, a, dfb = pl.pallas_call(
        _proj_body,
        in_specs=[pl.BlockSpec(memory_space=pltpu.VMEM)] * 7,
        out_specs=[pl.BlockSpec(memory_space=pltpu.VMEM)] * 3,
        out_shape=[
            jax.ShapeDtypeStruct((_BROWS, _D), f32),
            jax.ShapeDtypeStruct((_NH, _D), f32),
            jax.ShapeDtypeStruct((128, _D), f32),
        ],
        scratch_shapes=[pltpu.VMEM((_NT, _D), f32)],
    )(heads_ids[:, None], token_emb, w1at, w1bt, w1ct, b1r, emb_table)

    d127 = dfb[127:128]
    dflip = dfb[126::-1]
    dpad = jnp.concatenate(
        [jnp.tile(d127, (_G, 1)), dflip,
         jnp.tile(d127, (_DROWS - _G - 127, 1))], axis=0)

    heads_sent = jnp.take(sent_ids, heads_ids)
    starts = jnp.searchsorted(sent_ids, heads_sent, side='left').astype(jnp.int32)
    ends = jnp.searchsorted(sent_ids, heads_sent, side='right').astype(jnp.int32)
    lens = ends - starts
    a8s = starts // 8
    lmax = jnp.max(lens)[None].astype(jnp.int32)

    w2t = W2.T
    b2r = b2[None, :]
    w3t = W3.T
    b3r = b3[None, :]
    c1 = jnp.concatenate([conv1_w[:, :, k].T for k in range(3)], axis=0)
    c1b = conv1_b[None, :]
    c2 = jnp.concatenate([conv2_w[:, :, k].T for k in range(3)], axis=0)
    c2b = conv2_b[None, :]

    const = lambda h, *_: (0, 0)
    grid_spec = pltpu.PrefetchScalarGridSpec(
        num_scalar_prefetch=5,
        grid=(_NH,),
        in_specs=[
            pl.BlockSpec((1, 1, _D), lambda h, *_: (h, 0, 0)),
            pl.BlockSpec((_BROWS, _D), const),
            pl.BlockSpec((_DROWS, _D), const),
            pl.BlockSpec((1, _D), const),
            pl.BlockSpec((1, _D), const),
            pl.BlockSpec((_D, 256), const),
            pl.BlockSpec((1, 256), const),
            pl.BlockSpec((256, 64), const),
            pl.BlockSpec((1, 64), const),
            pl.BlockSpec((192, 4), const),
            pl.BlockSpec((1, 4), const),
            pl.BlockSpec((12, 2), const),
            pl.BlockSpec((1, 2), const),
        ],
        out_specs=pl.BlockSpec((1, _NT, 2), lambda h, *_: (h, 0, 0)),
    )
    out = pl.pallas_call(
        _span_body,
        grid_spec=grid_spec,
        out_shape=jax.ShapeDtypeStruct((_NH, _NT, 2), f32),
    )(a8s, starts, lens, heads_ids, lmax,
      a[:, None, :], bp, dpad, d127, b1r, w2t, b2r, w3t, b3r,
      c1, c1b, c2, c2b)
    return out
